# baseline (speedup 1.0000x reference)
import functools

import jax
import jax.numpy as jnp
from jax import lax
from jax.experimental import pallas as pl
from jax.experimental.pallas import tpu as pltpu
from jax.experimental.pallas import tpu_sc as plsc

VOCAB = 100000
EMBED = 64
BATCH = 1024
CTX = 50

_NC = 2
_NS = 16
_NW = _NC * _NS
_BPW = BATCH // _NW
_RPW = _BPW * CTX
_GCH = 80
_NCHUNK = _RPW // _GCH


def _rsqrt_newton(x):
    i = plsc.bitcast(x, jnp.int32)
    i = jnp.int32(0x5F3759DF) - lax.shift_right_logical(i, 1)
    y = plsc.bitcast(i, jnp.float32)
    for _ in range(3):
        y = y * (1.5 - 0.5 * x * y * y)
    return y


def _make_pool_kernel():
    mesh = plsc.VectorSubcoreMesh(core_axis_name="c", subcore_axis_name="s")

    @functools.partial(
        pl.kernel,
        out_type=jax.ShapeDtypeStruct((BATCH, EMBED), jnp.float32),
        mesh=mesh,
        compiler_params=pltpu.CompilerParams(
            needs_layout_passes=False, use_tc_tiling_on_sc=False
        ),
        scratch_types=[
            pltpu.VMEM((_RPW,), jnp.int32),
            pltpu.VMEM((_RPW, EMBED), jnp.float32),
            pltpu.VMEM((_BPW, EMBED), jnp.float32),
            pltpu.SemaphoreType.DMA,
        ],
    )
    def pool(table_hbm, idx_hbm, out_hbm, idx_v, rows_v, pool_v, sem):
        wid = lax.axis_index("s") * _NC + lax.axis_index("c")
        pltpu.sync_copy(idx_hbm.at[pl.ds(wid * _RPW, _RPW)], idx_v)
        copies = []
        for k in range(_NCHUNK):
            cp = pltpu.make_async_copy(
                table_hbm.at[idx_v.at[pl.ds(k * _GCH, _GCH)]],
                rows_v.at[pl.ds(k * _GCH, _GCH)],
                sem,
            )
            cp.start()
            copies.append(cp)
        for cp in copies:
            cp.wait()

        lanes = lax.iota(jnp.int32, 16)

        def norm_body(g, _):
            r0 = pl.multiple_of(g * 16, 16)
            rowidx = r0 + lanes

            def col_body(k, sqacc):
                colidx = jnp.full((16,), k, dtype=jnp.int32)
                v = plsc.load_gather(rows_v, [rowidx, colidx])
                return sqacc + v * v

            n2 = lax.fori_loop(0, EMBED, col_body, jnp.zeros((16,), jnp.float32))
            scale = jnp.where(n2 > 1.0, _rsqrt_newton(n2), 1.0)
            for i in range(16):
                s = scale[i]
                for j in range(EMBED // 16):
                    sl = (r0 + i, pl.ds(j * 16, 16))
                    rows_v[sl] = rows_v[sl] * s
            return 0

        lax.fori_loop(0, _RPW // 16, norm_body, 0)

        def row_body(b, _):
            def ctx_body(c, acc):
                r = b * CTX + c
                vs = [rows_v[r, pl.ds(j * 16, 16)] for j in range(EMBED // 16)]
                return tuple(a + v for a, v in zip(acc, vs))

            zero = jnp.zeros((16,), jnp.float32)
            acc = lax.fori_loop(0, CTX, ctx_body, (zero,) * (EMBED // 16))
            inv = jnp.float32(1.0 / CTX)
            for j in range(EMBED // 16):
                pool_v[b, pl.ds(j * 16, 16)] = acc[j] * inv
            return 0

        lax.fori_loop(0, _BPW, row_body, 0)
        pltpu.sync_copy(pool_v, out_hbm.at[pl.ds(wid * _BPW, _BPW)])

    return pool


_pool_kernel = _make_pool_kernel()

_VB = 1024


def _mm_body(p_ref, w_ref, b_ref, o_ref):
    o_ref[...] = (
        lax.dot_general(
            p_ref[...],
            w_ref[...],
            (((1,), (1,)), ((), ())),
            preferred_element_type=jnp.float32,
        )
        + b_ref[...]
    )


def _project(pooled, lin_w, lin_b2d):
    nblk = (VOCAB + _VB - 1) // _VB
    return pl.pallas_call(
        _mm_body,
        grid=(nblk,),
        in_specs=[
            pl.BlockSpec((BATCH, EMBED), lambda j: (0, 0)),
            pl.BlockSpec((_VB, EMBED), lambda j: (j, 0)),
            pl.BlockSpec((1, _VB), lambda j: (0, j)),
        ],
        out_specs=pl.BlockSpec((BATCH, _VB), lambda j: (0, j)),
        out_shape=jax.ShapeDtypeStruct((BATCH, VOCAB), jnp.float32),
        compiler_params=pltpu.CompilerParams(
            dimension_semantics=("parallel",),
        ),
    )(pooled, lin_w, lin_b2d)


def kernel(inputs_, emb_table, lin_w, lin_b):
    idx = inputs_.astype(jnp.int32).reshape(-1)
    pooled = _pool_kernel(emb_table, idx)
    return _project(pooled, lin_w, lin_b.reshape(1, VOCAB))


# speedup vs baseline: 1.0372x; 1.0372x over previous
import functools

import jax
import jax.numpy as jnp
from jax import lax
from jax.experimental import pallas as pl
from jax.experimental.pallas import tpu as pltpu
from jax.experimental.pallas import tpu_sc as plsc

VOCAB = 100000
EMBED = 64
BATCH = 1024
CTX = 50

_NC = 2
_NS = 16
_NW = _NC * _NS
_BPW = BATCH // _NW
_RPW = _BPW * CTX
_GCH = 80
_NCHUNK = _RPW // _GCH


def _rsqrt_newton(x):
    i = plsc.bitcast(x, jnp.int32)
    i = jnp.int32(0x5F3759DF) - lax.shift_right_logical(i, 1)
    y = plsc.bitcast(i, jnp.float32)
    for _ in range(3):
        y = y * (1.5 - 0.5 * x * y * y)
    return y


def _make_pool_kernel():
    mesh = plsc.VectorSubcoreMesh(core_axis_name="c", subcore_axis_name="s")

    @functools.partial(
        pl.kernel,
        out_type=jax.ShapeDtypeStruct((BATCH, EMBED), jnp.float32),
        mesh=mesh,
        compiler_params=pltpu.CompilerParams(
            needs_layout_passes=False, use_tc_tiling_on_sc=False
        ),
        scratch_types=[
            pltpu.VMEM((_RPW,), jnp.int32),
            pltpu.VMEM((_RPW, EMBED), jnp.float32),
            pltpu.VMEM((_BPW, EMBED), jnp.float32),
            pltpu.SemaphoreType.DMA,
        ],
    )
    def pool(table_hbm, idx_hbm, out_hbm, idx_v, rows_v, pool_v, sem):
        wid = lax.axis_index("s") * _NC + lax.axis_index("c")
        pltpu.sync_copy(idx_hbm.at[pl.ds(wid * _RPW, _RPW)], idx_v)
        copies = []
        for k in range(_NCHUNK):
            cp = pltpu.make_async_copy(
                table_hbm.at[idx_v.at[pl.ds(k * _GCH, _GCH)]],
                rows_v.at[pl.ds(k * _GCH, _GCH)],
                sem,
            )
            cp.start()
            copies.append(cp)
        for cp in copies:
            cp.wait()

        lanes = lax.iota(jnp.int32, 16)

        def norm_body(g, _):
            r0 = pl.multiple_of(g * 16, 16)
            rowidx = r0 + lanes

            def col_body(k, sqacc):
                colidx = jnp.full((16,), k, dtype=jnp.int32)
                v = plsc.load_gather(rows_v, [rowidx, colidx])
                return sqacc + v * v

            n2 = lax.fori_loop(0, EMBED, col_body, jnp.zeros((16,), jnp.float32))
            scale = jnp.where(n2 > 1.0, _rsqrt_newton(n2), 1.0)
            for i in range(16):
                s = scale[i]
                for j in range(EMBED // 16):
                    sl = (r0 + i, pl.ds(j * 16, 16))
                    rows_v[sl] = rows_v[sl] * s
            return 0

        lax.fori_loop(0, _RPW // 16, norm_body, 0)

        def row_body(b, _):
            def ctx_body(c, acc):
                r = b * CTX + c
                vs = [rows_v[r, pl.ds(j * 16, 16)] for j in range(EMBED // 16)]
                return tuple(a + v for a, v in zip(acc, vs))

            zero = jnp.zeros((16,), jnp.float32)
            acc = lax.fori_loop(0, CTX, ctx_body, (zero,) * (EMBED // 16))
            inv = jnp.float32(1.0 / CTX)
            for j in range(EMBED // 16):
                pool_v[b, pl.ds(j * 16, 16)] = acc[j] * inv
            return 0

        lax.fori_loop(0, _BPW, row_body, 0)
        pltpu.sync_copy(pool_v, out_hbm.at[pl.ds(wid * _BPW, _BPW)])

    return pool


_pool_kernel = _make_pool_kernel()

_VB = 4096


def _mm_body(p_ref, w_ref, b_ref, o_ref):
    o_ref[...] = (
        lax.dot_general(
            p_ref[...],
            w_ref[...],
            (((1,), (1,)), ((), ())),
            preferred_element_type=jnp.float32,
        )
        + b_ref[...]
    )


def _project(pooled, lin_w, lin_b2d):
    nblk = (VOCAB + _VB - 1) // _VB
    return pl.pallas_call(
        _mm_body,
        grid=(nblk,),
        in_specs=[
            pl.BlockSpec((BATCH, EMBED), lambda j: (0, 0)),
            pl.BlockSpec((_VB, EMBED), lambda j: (j, 0)),
            pl.BlockSpec((1, _VB), lambda j: (0, j)),
        ],
        out_specs=pl.BlockSpec((BATCH, _VB), lambda j: (0, j)),
        out_shape=jax.ShapeDtypeStruct((BATCH, VOCAB), jnp.float32),
        compiler_params=pltpu.CompilerParams(
            dimension_semantics=("parallel",),
        ),
    )(pooled, lin_w, lin_b2d)


def kernel(inputs_, emb_table, lin_w, lin_b):
    idx = inputs_.astype(jnp.int32).reshape(-1)
    pooled = _pool_kernel(emb_table, idx)
    return _project(pooled, lin_w, lin_b.reshape(1, VOCAB))
